# baseline (speedup 1.0000x reference)
import functools

import jax
import jax.numpy as jnp
from jax import lax
from jax.experimental import pallas as pl
from jax.experimental.pallas import tpu as pltpu
from jax.experimental.pallas import tpu_sc as plsc

B = 16384
D = 32
F = 16
NC = 2
NS = 16
L = 16
NW = NC * NS
BPW = B // NW
CHUNK = 128
NG = BPW // L

N_TAB = 1000000
SW = 512
NSTRIP = N_TAB // SW
NTAIL = N_TAB - NSTRIP * SW

_mesh = plsc.VectorSubcoreMesh(
    core_axis_name="c", subcore_axis_name="s", num_cores=NC, num_subcores=NS
)

_CP_COMPACT = pltpu.CompilerParams(needs_layout_passes=False)
_CP_UNTILED = pltpu.CompilerParams(
    needs_layout_passes=False, use_tc_tiling_on_sc=False
)


def _make_relayout(dt):
    upr = 128 // dt
    rps = SW // upr
    nseg = 128 // L
    tail_rows = NTAIL // upr

    def body(xt_hbm, tail_hbm, out_hbm, xs, os, ts, sem):
        wid = lax.axis_index("s") * NC + lax.axis_index("c")
        lane = lax.iota(jnp.int32, L)
        rvs = [lane + 16 * h for h in range(dt // L)]

        def fire(strip, buf):
            col0 = pl.multiple_of(strip * SW, SW)
            return pltpu.async_copy(
                xt_hbm.at[:, pl.ds(col0, SW)], xs.at[buf], sem)

        fire(wid, 0)

        def step(k, carry):
            strip = wid + NW * k

            @pl.when(strip + NW < NSTRIP)
            def _():
                fire(strip + NW, (k + 1) & 1)

            @pl.when(strip < NSTRIP)
            def _():
                pltpu.make_async_copy(
                    xt_hbm.at[:, pl.ds(0, SW)], xs.at[k & 1], sem).wait()
                x = xs.at[k & 1]

                def row(r, rcarry):
                    jb = upr * r
                    for w in range(nseg):
                        j = jb + (w // (dt // L))
                        rv = rvs[w % (dt // L)]
                        seg = plsc.load_gather(x, [rv, jnp.full((L,), 1, jnp.int32) * j])
                        os[r, pl.ds(16 * w, L)] = seg
                    return rcarry

                lax.fori_loop(0, rps, row, 0)
                row0 = pl.multiple_of(strip * rps, 8)
                pltpu.sync_copy(os, out_hbm.at[pl.ds(row0, rps), :])
            return carry

        lax.fori_loop(0, (NSTRIP + NW - 1) // NW, step, 0)

        @pl.when(wid == 0)
        def _():
            pltpu.sync_copy(tail_hbm, ts)
            row0 = NSTRIP * rps
            pltpu.sync_copy(ts, out_hbm.at[pl.ds(row0, tail_rows), :])

    return functools.partial(
        pl.kernel,
        out_type=jax.ShapeDtypeStruct((N_TAB * dt // 128, 128), jnp.float32),
        mesh=_mesh,
        scratch_types=[
            pltpu.VMEM((2, dt, SW), jnp.float32),
            pltpu.VMEM((rps, 128), jnp.float32),
            pltpu.VMEM((NTAIL // upr, 128), jnp.float32),
            pltpu.SemaphoreType.DMA,
        ],
        compiler_params=_CP_COMPACT,
    )(body)


_relayout32 = _make_relayout(32)
_relayout16 = _make_relayout(16)


HB = BPW // 2
HG = HB // L


def _dot_body(u_hbm, i_hbm, p2_hbm, q2_hbm, f2_hbm, fw_hbm, out_hbm,
              uv, iv, uh2, ih2, ih3, pv, qv, fv, fwv, outv, sem):
    wid = lax.axis_index("s") * NC + lax.axis_index("c")
    base = pl.multiple_of(wid * BPW, BPW)

    pltpu.sync_copy(u_hbm.at[pl.ds(base, BPW)], uv)
    pltpu.sync_copy(i_hbm.at[pl.ds(base, BPW)], iv)
    pltpu.sync_copy(fw_hbm, fwv)

    lane = lax.iota(jnp.int32, L)
    mask3 = jnp.full((L,), 3, jnp.int32)
    mask7 = jnp.full((L,), 7, jnp.int32)

    for h in range(2):
        for t in range(HB // L):
            s = pl.ds(t * L, L)
            sh = pl.ds(h * HB + t * L, L)
            uh2[s] = jax.lax.shift_right_logical(uv[sh], 2)
            ih2[s] = jax.lax.shift_right_logical(iv[sh], 2)
            ih3[s] = jax.lax.shift_right_logical(iv[sh], 3)

        cps = []
        for j in range(HB // CHUNK):
            r = pl.ds(j * CHUNK, CHUNK)
            cps.append(pltpu.async_copy(p2_hbm.at[uh2.at[r]], pv.at[r], sem))
            cps.append(pltpu.async_copy(q2_hbm.at[ih2.at[r]], qv.at[r], sem))
            cps.append(pltpu.async_copy(f2_hbm.at[ih3.at[r]], fv.at[r], sem))
        for c in cps:
            c.wait()

        def group(g, carry):
            row0 = pl.multiple_of(g * L, L)
            uvec = uv[pl.ds(h * HB + row0, L)]
            ivec = iv[pl.ds(h * HB + row0, L)]
            ridx = row0 + lane
            ubase = (uvec & mask3) << 5
            ibase = (ivec & mask3) << 5
            fbase = (ivec & mask7) << 4
            feats = [plsc.load_gather(
                         fv, [ridx, fbase + jnp.full((L,), f, jnp.int32)])
                     for f in range(F)]
            acc = jnp.zeros((L,), jnp.float32)
            for d in range(D):
                dd = jnp.full((L,), d, jnp.int32)
                p_d = plsc.load_gather(pv, [ridx, ubase + dd])
                q_d = plsc.load_gather(qv, [ridx, ibase + dd])
                fwd = fwv[d, :]
                for f in range(F):
                    q_d = q_d + feats[f] * fwd[f]
                acc = acc + p_d * q_d
            outv[pl.ds(h * HB + row0, L)] = acc
            return carry

        lax.fori_loop(0, HG, group, 0)

    pltpu.sync_copy(outv, out_hbm.at[pl.ds(base, BPW)])


_dot_call = functools.partial(
    pl.kernel,
    out_type=jax.ShapeDtypeStruct((B,), jnp.float32),
    mesh=_mesh,
    scratch_types=[
        pltpu.VMEM((BPW,), jnp.int32),
        pltpu.VMEM((BPW,), jnp.int32),
        pltpu.VMEM((HB,), jnp.int32),
        pltpu.VMEM((HB,), jnp.int32),
        pltpu.VMEM((HB,), jnp.int32),
        pltpu.VMEM((HB, 128), jnp.float32),
        pltpu.VMEM((HB, 128), jnp.float32),
        pltpu.VMEM((HB, 128), jnp.float32),
        pltpu.VMEM((D, F), jnp.float32),
        pltpu.VMEM((BPW,), jnp.float32),
        pltpu.SemaphoreType.DMA,
    ],
    compiler_params=_CP_COMPACT,
)(_dot_body)


def _bias_body(u_hbm, i_hbm, bu_hbm, bi_hbm, mu16_hbm, part_hbm, out_hbm,
               uv, iv, ubh, ibh, buv, biv, muv, pv, outv, sem):
    wid = lax.axis_index("s") * NC + lax.axis_index("c")
    base = pl.multiple_of(wid * BPW, BPW)

    pltpu.sync_copy(u_hbm.at[pl.ds(base, BPW)], uv)
    pltpu.sync_copy(i_hbm.at[pl.ds(base, BPW)], iv)
    pltpu.sync_copy(mu16_hbm, muv)
    pltpu.sync_copy(part_hbm.at[pl.ds(base, BPW)], pv)

    for t in range(BPW // L):
        s = pl.ds(t * L, L)
        ubh[s] = jax.lax.shift_right_logical(uv[s], 4)
        ibh[s] = jax.lax.shift_right_logical(iv[s], 4)

    cps = []
    for j in range(BPW // CHUNK):
        r = pl.ds(j * CHUNK, CHUNK)
        cps.append(pltpu.async_copy(bu_hbm.at[ubh.at[r]], buv.at[r], sem))
        cps.append(pltpu.async_copy(bi_hbm.at[ibh.at[r]], biv.at[r], sem))
    for c in cps:
        c.wait()

    lane = lax.iota(jnp.int32, L)
    mu_vec = muv[...]
    mask15 = jnp.full((L,), 15, jnp.int32)

    def group(g, carry):
        row0 = pl.multiple_of(g * L, L)
        ridx = row0 + lane
        uvec = uv[pl.ds(row0, L)]
        ivec = iv[pl.ds(row0, L)]
        bu_g = plsc.load_gather(buv, [ridx, uvec & mask15])
        bi_g = plsc.load_gather(biv, [ridx, ivec & mask15])
        outv[pl.ds(row0, L)] = pv[pl.ds(row0, L)] + bu_g + bi_g + mu_vec
        return carry

    lax.fori_loop(0, NG, group, 0)
    pltpu.sync_copy(outv, out_hbm.at[pl.ds(base, BPW)])


_bias_call = functools.partial(
    pl.kernel,
    out_type=jax.ShapeDtypeStruct((B,), jnp.float32),
    mesh=_mesh,
    scratch_types=[
        pltpu.VMEM((BPW,), jnp.int32),
        pltpu.VMEM((BPW,), jnp.int32),
        pltpu.VMEM((BPW,), jnp.int32),
        pltpu.VMEM((BPW,), jnp.int32),
        pltpu.VMEM((BPW, L), jnp.float32),
        pltpu.VMEM((BPW, L), jnp.float32),
        pltpu.VMEM((L,), jnp.float32),
        pltpu.VMEM((BPW,), jnp.float32),
        pltpu.VMEM((BPW,), jnp.float32),
        pltpu.SemaphoreType.DMA,
    ],
    compiler_params=_CP_UNTILED,
)(_bias_body)


def kernel(u, i, P, Q, bu, bi, mu, F_w, item_features):
    nu = P.shape[0]
    ni = Q.shape[0]
    u32 = u.astype(jnp.int32)
    i32 = i.astype(jnp.int32)
    mu16 = jnp.broadcast_to(mu.astype(jnp.float32), (L,))
    tail = NSTRIP * SW
    p2 = _relayout32(P.T, P[tail:, :].reshape(-1, 128))
    q2 = _relayout32(Q.T, Q[tail:, :].reshape(-1, 128))
    f2 = _relayout16(item_features.T, item_features[tail:, :].reshape(-1, 128))
    part = _dot_call(u32, i32, p2, q2, f2, F_w)
    return _bias_call(
        u32, i32,
        bu.reshape(nu // L, L), bi.reshape(ni // L, L),
        mu16, part,
    )


# speedup vs baseline: 1.3332x; 1.3332x over previous
import functools

import jax
import jax.numpy as jnp
from jax import lax
from jax.experimental import pallas as pl
from jax.experimental.pallas import tpu as pltpu
from jax.experimental.pallas import tpu_sc as plsc

B = 16384
D = 32
F = 16
NC = 2
NS = 16
L = 16
NW = NC * NS
BPW = B // NW
CHUNK = 128
NG = BPW // L

N_TAB = 1000000
SW = 512
NSTRIP = N_TAB // SW
NTAIL = N_TAB - NSTRIP * SW

_mesh = plsc.VectorSubcoreMesh(
    core_axis_name="c", subcore_axis_name="s", num_cores=NC, num_subcores=NS
)

_CP_COMPACT = pltpu.CompilerParams(needs_layout_passes=False)
_CP_UNTILED = pltpu.CompilerParams(
    needs_layout_passes=False, use_tc_tiling_on_sc=False
)


def _make_relayout(dt):
    upr = 128 // dt
    rps = SW // upr
    nseg = 128 // L
    tail_rows = NTAIL // upr

    def body(xt_hbm, tail_hbm, out_hbm, xs, os, ts, sem):
        wid = lax.axis_index("s") * NC + lax.axis_index("c")
        lane = lax.iota(jnp.int32, L)
        rvs = [lane + 16 * h for h in range(dt // L)]

        def fire(strip, buf):
            col0 = pl.multiple_of(strip * SW, SW)
            return pltpu.async_copy(
                xt_hbm.at[:, pl.ds(col0, SW)], xs.at[buf], sem)

        fire(wid, 0)

        def step(k, carry):
            strip = wid + NW * k

            @pl.when(strip + NW < NSTRIP)
            def _():
                fire(strip + NW, (k + 1) & 1)

            @pl.when(strip < NSTRIP)
            def _():
                pltpu.make_async_copy(
                    xt_hbm.at[:, pl.ds(0, SW)], xs.at[k & 1], sem).wait()
                x = xs.at[k & 1]

                def row(r8, rcarry):
                    r0 = r8 * 8
                    segs = []
                    for rr in range(8):
                        jb = upr * (r0 + rr)
                        for w in range(nseg):
                            j = jb + (w // (dt // L))
                            rv = rvs[w % (dt // L)]
                            segs.append(plsc.load_gather(
                                x, [rv, jnp.full((L,), 1, jnp.int32) * j]))
                    for rr in range(8):
                        for w in range(nseg):
                            os[r0 + rr, pl.ds(16 * w, L)] = segs[rr * nseg + w]
                    return rcarry

                lax.fori_loop(0, rps // 8, row, 0)
                row0 = pl.multiple_of(strip * rps, 8)
                pltpu.sync_copy(os, out_hbm.at[pl.ds(row0, rps), :])
            return carry

        lax.fori_loop(0, (NSTRIP + NW - 1) // NW, step, 0)

        @pl.when(wid == 0)
        def _():
            pltpu.sync_copy(tail_hbm, ts)
            row0 = NSTRIP * rps
            pltpu.sync_copy(ts, out_hbm.at[pl.ds(row0, tail_rows), :])

    return functools.partial(
        pl.kernel,
        out_type=jax.ShapeDtypeStruct((N_TAB * dt // 128, 128), jnp.float32),
        mesh=_mesh,
        scratch_types=[
            pltpu.VMEM((2, dt, SW), jnp.float32),
            pltpu.VMEM((rps, 128), jnp.float32),
            pltpu.VMEM((NTAIL // upr, 128), jnp.float32),
            pltpu.SemaphoreType.DMA,
        ],
        compiler_params=_CP_COMPACT,
    )(body)


_relayout32 = _make_relayout(32)
_relayout16 = _make_relayout(16)


HB = BPW // 2
HG = HB // L


def _dot_body(u_hbm, i_hbm, p2_hbm, q2_hbm, f2_hbm, fw_hbm, out_hbm,
              uv, iv, uh2, ih2, ih3, pv, qv, fv, fwv, outv, sem):
    wid = lax.axis_index("s") * NC + lax.axis_index("c")
    base = pl.multiple_of(wid * BPW, BPW)

    pltpu.sync_copy(u_hbm.at[pl.ds(base, BPW)], uv)
    pltpu.sync_copy(i_hbm.at[pl.ds(base, BPW)], iv)
    pltpu.sync_copy(fw_hbm, fwv)

    lane = lax.iota(jnp.int32, L)
    mask3 = jnp.full((L,), 3, jnp.int32)
    mask7 = jnp.full((L,), 7, jnp.int32)

    for h in range(2):
        for t in range(HB // L):
            s = pl.ds(t * L, L)
            sh = pl.ds(h * HB + t * L, L)
            uh2[s] = jax.lax.shift_right_logical(uv[sh], 2)
            ih2[s] = jax.lax.shift_right_logical(iv[sh], 2)
            ih3[s] = jax.lax.shift_right_logical(iv[sh], 3)

        cps = []
        for j in range(HB // CHUNK):
            r = pl.ds(j * CHUNK, CHUNK)
            cps.append(pltpu.async_copy(p2_hbm.at[uh2.at[r]], pv.at[r], sem))
            cps.append(pltpu.async_copy(q2_hbm.at[ih2.at[r]], qv.at[r], sem))
            cps.append(pltpu.async_copy(f2_hbm.at[ih3.at[r]], fv.at[r], sem))
        for c in cps:
            c.wait()

        def group(g, carry):
            row0 = pl.multiple_of(g * L, L)
            uvec = uv[pl.ds(h * HB + row0, L)]
            ivec = iv[pl.ds(h * HB + row0, L)]
            ridx = row0 + lane
            ubase = (uvec & mask3) << 5
            ibase = (ivec & mask3) << 5
            fbase = (ivec & mask7) << 4
            feats = [plsc.load_gather(
                         fv, [ridx, fbase + jnp.full((L,), f, jnp.int32)])
                     for f in range(F)]
            acc = jnp.zeros((L,), jnp.float32)
            for d in range(D):
                dd = jnp.full((L,), d, jnp.int32)
                p_d = plsc.load_gather(pv, [ridx, ubase + dd])
                q_d = plsc.load_gather(qv, [ridx, ibase + dd])
                fwd = fwv[d, :]
                for f in range(F):
                    q_d = q_d + feats[f] * fwd[f]
                acc = acc + p_d * q_d
            outv[pl.ds(h * HB + row0, L)] = acc
            return carry

        lax.fori_loop(0, HG, group, 0)

    pltpu.sync_copy(outv, out_hbm.at[pl.ds(base, BPW)])


_dot_call = functools.partial(
    pl.kernel,
    out_type=jax.ShapeDtypeStruct((B,), jnp.float32),
    mesh=_mesh,
    scratch_types=[
        pltpu.VMEM((BPW,), jnp.int32),
        pltpu.VMEM((BPW,), jnp.int32),
        pltpu.VMEM((HB,), jnp.int32),
        pltpu.VMEM((HB,), jnp.int32),
        pltpu.VMEM((HB,), jnp.int32),
        pltpu.VMEM((HB, 128), jnp.float32),
        pltpu.VMEM((HB, 128), jnp.float32),
        pltpu.VMEM((HB, 128), jnp.float32),
        pltpu.VMEM((D, F), jnp.float32),
        pltpu.VMEM((BPW,), jnp.float32),
        pltpu.SemaphoreType.DMA,
    ],
    compiler_params=_CP_COMPACT,
)(_dot_body)


def _bias_body(u_hbm, i_hbm, bu_hbm, bi_hbm, mu16_hbm, part_hbm, out_hbm,
               uv, iv, ubh, ibh, buv, biv, muv, pv, outv, sem):
    wid = lax.axis_index("s") * NC + lax.axis_index("c")
    base = pl.multiple_of(wid * BPW, BPW)

    pltpu.sync_copy(u_hbm.at[pl.ds(base, BPW)], uv)
    pltpu.sync_copy(i_hbm.at[pl.ds(base, BPW)], iv)
    pltpu.sync_copy(mu16_hbm, muv)
    pltpu.sync_copy(part_hbm.at[pl.ds(base, BPW)], pv)

    for t in range(BPW // L):
        s = pl.ds(t * L, L)
        ubh[s] = jax.lax.shift_right_logical(uv[s], 4)
        ibh[s] = jax.lax.shift_right_logical(iv[s], 4)

    cps = []
    for j in range(BPW // CHUNK):
        r = pl.ds(j * CHUNK, CHUNK)
        cps.append(pltpu.async_copy(bu_hbm.at[ubh.at[r]], buv.at[r], sem))
        cps.append(pltpu.async_copy(bi_hbm.at[ibh.at[r]], biv.at[r], sem))
    for c in cps:
        c.wait()

    lane = lax.iota(jnp.int32, L)
    mu_vec = muv[...]
    mask15 = jnp.full((L,), 15, jnp.int32)

    def group(g, carry):
        row0 = pl.multiple_of(g * L, L)
        ridx = row0 + lane
        uvec = uv[pl.ds(row0, L)]
        ivec = iv[pl.ds(row0, L)]
        bu_g = plsc.load_gather(buv, [ridx, uvec & mask15])
        bi_g = plsc.load_gather(biv, [ridx, ivec & mask15])
        outv[pl.ds(row0, L)] = pv[pl.ds(row0, L)] + bu_g + bi_g + mu_vec
        return carry

    lax.fori_loop(0, NG, group, 0)
    pltpu.sync_copy(outv, out_hbm.at[pl.ds(base, BPW)])


_bias_call = functools.partial(
    pl.kernel,
    out_type=jax.ShapeDtypeStruct((B,), jnp.float32),
    mesh=_mesh,
    scratch_types=[
        pltpu.VMEM((BPW,), jnp.int32),
        pltpu.VMEM((BPW,), jnp.int32),
        pltpu.VMEM((BPW,), jnp.int32),
        pltpu.VMEM((BPW,), jnp.int32),
        pltpu.VMEM((BPW, L), jnp.float32),
        pltpu.VMEM((BPW, L), jnp.float32),
        pltpu.VMEM((L,), jnp.float32),
        pltpu.VMEM((BPW,), jnp.float32),
        pltpu.VMEM((BPW,), jnp.float32),
        pltpu.SemaphoreType.DMA,
    ],
    compiler_params=_CP_UNTILED,
)(_bias_body)


def kernel(u, i, P, Q, bu, bi, mu, F_w, item_features):
    nu = P.shape[0]
    ni = Q.shape[0]
    u32 = u.astype(jnp.int32)
    i32 = i.astype(jnp.int32)
    mu16 = jnp.broadcast_to(mu.astype(jnp.float32), (L,))
    tail = NSTRIP * SW
    p2 = _relayout32(P.T, P[tail:, :].reshape(-1, 128))
    q2 = _relayout32(Q.T, Q[tail:, :].reshape(-1, 128))
    f2 = _relayout16(item_features.T, item_features[tail:, :].reshape(-1, 128))
    part = _dot_call(u32, i32, p2, q2, f2, F_w)
    return _bias_call(
        u32, i32,
        bu.reshape(nu // L, L), bi.reshape(ni // L, L),
        mu16, part,
    )


# speedup vs baseline: 1.7129x; 1.2848x over previous
import functools

import jax
import jax.numpy as jnp
from jax import lax
from jax.experimental import pallas as pl
from jax.experimental.pallas import tpu as pltpu
from jax.experimental.pallas import tpu_sc as plsc

B = 16384
D = 32
F = 16
NC = 2
NS = 16
L = 16
NW = NC * NS
BPW = B // NW
CHUNK = 128
NCHUNK = BPW // CHUNK
NG = BPW // L

_mesh = plsc.VectorSubcoreMesh(
    core_axis_name="c", subcore_axis_name="s", num_cores=NC, num_subcores=NS
)


def _body(u_hbm, i_hbm, p_hbm, q_hbm, bu_hbm, bi_hbm, mu16_hbm, fw_hbm,
          feat_hbm, out_hbm, uv, iv, ubh, ibh, pv, qv, fv, buv, biv, fwv,
          muv, outv, sem):
    wid = lax.axis_index("s") * NC + lax.axis_index("c")
    base = pl.multiple_of(wid * BPW, BPW)

    pltpu.sync_copy(u_hbm.at[pl.ds(base, BPW)], uv)
    pltpu.sync_copy(i_hbm.at[pl.ds(base, BPW)], iv)
    pltpu.sync_copy(fw_hbm, fwv)
    pltpu.sync_copy(mu16_hbm, muv)

    for t in range(BPW // L):
        s = pl.ds(t * L, L)
        ubh[s] = jax.lax.shift_right_logical(uv[s], 4)
        ibh[s] = jax.lax.shift_right_logical(iv[s], 4)

    cps = []
    for j in range(NCHUNK):
        r = pl.ds(j * CHUNK, CHUNK)
        cps.append(pltpu.async_copy(p_hbm.at[uv.at[r]], pv.at[r], sem))
        cps.append(pltpu.async_copy(q_hbm.at[iv.at[r]], qv.at[r], sem))
        cps.append(pltpu.async_copy(feat_hbm.at[iv.at[r]], fv.at[r], sem))
        cps.append(pltpu.async_copy(bu_hbm.at[ubh.at[r]], buv.at[r], sem))
        cps.append(pltpu.async_copy(bi_hbm.at[ibh.at[r]], biv.at[r], sem))
    for c in cps:
        c.wait()

    lane = lax.iota(jnp.int32, L)
    mu_vec = muv[...]
    mask15 = jnp.full((L,), 15, jnp.int32)

    def group(g, carry):
        row0 = pl.multiple_of(g * L, L)
        ridx = row0 + lane
        uvec = uv[pl.ds(row0, L)]
        ivec = iv[pl.ds(row0, L)]
        bu_g = plsc.load_gather(buv, [ridx, uvec & mask15])
        bi_g = plsc.load_gather(biv, [ridx, ivec & mask15])
        feats = [plsc.load_gather(fv, [ridx, jnp.full((L,), f, jnp.int32)])
                 for f in range(F)]
        acc = bu_g + bi_g + mu_vec
        for d in range(D):
            dd = jnp.full((L,), d, jnp.int32)
            p_d = plsc.load_gather(pv, [ridx, dd])
            q_d = plsc.load_gather(qv, [ridx, dd])
            fwd = fwv[d, :]
            for f in range(F):
                q_d = q_d + feats[f] * fwd[f]
            acc = acc + p_d * q_d
        outv[pl.ds(row0, L)] = acc
        return carry

    lax.fori_loop(0, NG, group, 0)
    pltpu.sync_copy(outv, out_hbm.at[pl.ds(base, BPW)])


_hybrid_mf_sc = functools.partial(
    pl.kernel,
    out_type=jax.ShapeDtypeStruct((B,), jnp.float32),
    mesh=_mesh,
    scratch_types=[
        pltpu.VMEM((BPW,), jnp.int32),
        pltpu.VMEM((BPW,), jnp.int32),
        pltpu.VMEM((BPW,), jnp.int32),
        pltpu.VMEM((BPW,), jnp.int32),
        pltpu.VMEM((BPW, D), jnp.float32),
        pltpu.VMEM((BPW, D), jnp.float32),
        pltpu.VMEM((BPW, F), jnp.float32),
        pltpu.VMEM((BPW, L), jnp.float32),
        pltpu.VMEM((BPW, L), jnp.float32),
        pltpu.VMEM((D, F), jnp.float32),
        pltpu.VMEM((L,), jnp.float32),
        pltpu.VMEM((BPW,), jnp.float32),
        pltpu.SemaphoreType.DMA,
    ],
    compiler_params=pltpu.CompilerParams(
        needs_layout_passes=False, use_tc_tiling_on_sc=False
    ),
)(_body)


def kernel(u, i, P, Q, bu, bi, mu, F_w, item_features):
    nu = P.shape[0]
    ni = Q.shape[0]
    mu16 = jnp.broadcast_to(mu.astype(jnp.float32), (L,))
    return _hybrid_mf_sc(
        u.astype(jnp.int32), i.astype(jnp.int32),
        P, Q,
        bu.reshape(nu // L, L), bi.reshape(ni // L, L),
        mu16, F_w, item_features,
    )
